# Initial kernel scaffold; baseline (speedup 1.0000x reference)
import jax
import jax.numpy as jnp
from jax.experimental import pallas as pl


def kernel(task_score_head, task_score_labels, task_agn_idx):
    raise NotImplementedError("write your pallas kernel here")


# speedup vs baseline: 2.7060x; 2.7060x over previous
import functools

import jax
import jax.numpy as jnp
from jax import lax
from jax.experimental import pallas as pl
from jax.experimental.pallas import tpu as pltpu
from jax.experimental.pallas import tpu_sc as plsc

N = 1048576
K = 4096
NC = 2
NS = 16
NW = NC * NS
L = 16
M = N // NW
CH = 8192
NCH = M // CH
NB = 2048
NBV = NB // L

_mesh = plsc.VectorSubcoreMesh(core_axis_name="c", subcore_axis_name="s")
_cparams = pltpu.CompilerParams(needs_layout_passes=False)


def _bce16(x, y):
    e = jnp.exp(-jnp.abs(x))
    z = e / (e + 2.0)
    p = z * z
    l1p = 2.0 * z * (1.0 + p * (0.33333333 + p * (0.2 + p * 0.14285714)))
    return jnp.maximum(x, 0.0) - x * y + l1p


def _keybins(bce):
    key = plsc.bitcast(bce, jnp.int32)
    sh20 = jnp.full((L,), 20, jnp.int32)
    sh9 = jnp.full((L,), 9, jnp.int32)
    m11 = jnp.full((L,), 0x7FF, jnp.int32)
    b1 = lax.shift_right_logical(key, sh20)
    b2 = jnp.bitwise_and(lax.shift_right_logical(key, sh9), m11)
    return b1, b2


def _zero_hists(hc, hs):
    zero16 = jnp.zeros((L,), jnp.float32)

    def z(i, _):
        hc[pl.ds(i * L, L)] = zero16
        hs[pl.ds(i * L, L)] = zero16
        return 0

    lax.fori_loop(0, NB * L // L, z, 0)


def _merge_lanes_and_store(hc, hs, mc, ms, out_c, out_s, wid):
    def merge(j, _):
        accc = jnp.zeros((L,), jnp.float32)
        accs = jnp.zeros((L,), jnp.float32)
        for lane in range(L):
            accc = accc + hc[pl.ds(lane * NB + j * L, L)]
            accs = accs + hs[pl.ds(lane * NB + j * L, L)]
        mc[pl.ds(j * L, L)] = accc
        ms[pl.ds(j * L, L)] = accs
        return 0

    lax.fori_loop(0, NBV, merge, 0)
    pltpu.sync_copy(mc, out_c.at[pl.ds(wid * NB, NB)])
    pltpu.sync_copy(ms, out_s.at[pl.ds(wid * NB, NB)])


def _find_bin(mc, threshold):
    def body(jj, carry):
        cnt_acc, sum_carry = carry
        j = NBV - 1 - jj
        v = mc[pl.ds(j * L, L)]
        sfx = lax.rev(jnp.cumsum(lax.rev(v, (0,))), (0,)) + sum_carry
        ge = sfx >= threshold
        cnt_acc = cnt_acc + plsc.all_reduce_population_count(ge)
        return cnt_acc, sum_carry + jnp.sum(v)

    cnt, _ = lax.fori_loop(
        0, NBV, body, (jnp.zeros((L,), jnp.int32), jnp.float32(0.0))
    )
    return cnt - 1


def _pass1_body(x_hbm, y_hbm, h1c_hbm, h1s_hbm, xbuf, ybuf, hc, hs, mc, ms):
    wid = lax.axis_index("s") * NC + lax.axis_index("c")
    base = wid * M
    lane = lax.iota(jnp.int32, L)
    ones = jnp.ones((L,), jnp.float32)
    _zero_hists(hc, hs)
    for ch in range(NCH):
        pltpu.sync_copy(x_hbm.at[pl.ds(base + ch * CH, CH)], xbuf)
        pltpu.sync_copy(y_hbm.at[pl.ds(base + ch * CH, CH)], ybuf)

        def body(i, _):
            x = xbuf[pl.ds(i * L, L)]
            y = ybuf[pl.ds(i * L, L)]
            bce = _bce16(x, y)
            b1, _b2 = _keybins(bce)
            idx = b1 + lane * NB
            plsc.addupdate_scatter(hc, [idx], ones)
            plsc.addupdate_scatter(hs, [idx], bce)
            return 0

        lax.fori_loop(0, CH // L, body, 0)
    _merge_lanes_and_store(hc, hs, mc, ms, h1c_hbm, h1s_hbm, wid)


def _merge_rows(src_hbm, buf, dst, n_rows):
    zero16 = jnp.zeros((L,), jnp.float32)

    def z(i, _):
        dst[pl.ds(i * L, L)] = zero16
        return 0

    lax.fori_loop(0, NBV, z, 0)
    rows_per_chunk = CH // NB
    for g in range(n_rows // rows_per_chunk):
        pltpu.sync_copy(src_hbm.at[pl.ds(g * CH, CH)], buf)

        def acc(j, _):
            jd = j - (j // NBV) * NBV
            dst[pl.ds(jd * L, L)] = dst[pl.ds(jd * L, L)] + buf[pl.ds(j * L, L)]
            return 0

        lax.fori_loop(0, CH // L, acc, 0)


def _pass2_body(x_hbm, y_hbm, h1c_hbm, h2c_hbm, h2s_hbm,
                xbuf, ybuf, hc, hs, mc, ms):
    wid = lax.axis_index("s") * NC + lax.axis_index("c")
    base = wid * M
    lane = lax.iota(jnp.int32, L)
    ones = jnp.ones((L,), jnp.float32)
    _merge_rows(h1c_hbm, xbuf, mc, NW)
    b1_splat = _find_bin(mc, jnp.float32(float(K)))
    _zero_hists(hc, hs)
    for ch in range(NCH):
        pltpu.sync_copy(x_hbm.at[pl.ds(base + ch * CH, CH)], xbuf)
        pltpu.sync_copy(y_hbm.at[pl.ds(base + ch * CH, CH)], ybuf)

        def body(i, _):
            x = xbuf[pl.ds(i * L, L)]
            y = ybuf[pl.ds(i * L, L)]
            bce = _bce16(x, y)
            b1, b2 = _keybins(bce)
            mask = b1 == b1_splat
            idx = b2 + lane * NB
            plsc.addupdate_scatter(hc, [idx], ones, mask=mask)
            plsc.addupdate_scatter(hs, [idx], bce, mask=mask)
            return 0

        lax.fori_loop(0, CH // L, body, 0)
    _merge_lanes_and_store(hc, hs, mc, ms, h2c_hbm, h2s_hbm, wid)


def _masked_sums(mc, ms, bin_splat):
    lane = lax.iota(jnp.int32, L)

    def body(j, carry):
        cgt, sgt, ceq, seq = carry
        ids = lane + j * L
        vc = mc[pl.ds(j * L, L)]
        vs = ms[pl.ds(j * L, L)]
        gt = ids > bin_splat
        eq = ids == bin_splat
        zero = jnp.zeros((L,), jnp.float32)
        return (cgt + jnp.where(gt, vc, zero), sgt + jnp.where(gt, vs, zero),
                ceq + jnp.where(eq, vc, zero), seq + jnp.where(eq, vs, zero))

    z = jnp.zeros((L,), jnp.float32)
    cgt, sgt, ceq, seq = lax.fori_loop(0, NBV, body, (z, z, z, z))
    return jnp.sum(cgt), jnp.sum(sgt), jnp.sum(ceq), jnp.sum(seq)


def _pass3_body(h1c_hbm, h1s_hbm, h2c_hbm, h2s_hbm, loss_hbm,
                xbuf, mc, ms, obuf):
    wid = lax.axis_index("s") * NC + lax.axis_index("c")

    @pl.when(wid == 0)
    def _():
        _merge_rows(h1c_hbm, xbuf, mc, NW)
        _merge_rows(h1s_hbm, xbuf, ms, NW)
        b1_splat = _find_bin(mc, jnp.float32(float(K)))
        c_ab, s_ab, _c1, _s1 = _masked_sums(mc, ms, b1_splat)
        _merge_rows(h2c_hbm, xbuf, mc, NW)
        _merge_rows(h2s_hbm, xbuf, ms, NW)
        t2 = jnp.float32(float(K)) - c_ab
        b2_splat = _find_bin(mc, t2)
        c_hi2, s_hi2, c_str, s_str = _masked_sums(mc, ms, b2_splat)
        ones = jnp.ones((L,), jnp.float32)
        kf = jnp.full((L,), float(K), jnp.float32)
        c_hi = ones * c_ab + ones * c_hi2
        s_hi = ones * s_ab + ones * s_hi2
        borrow = (kf - c_hi) * (ones * s_str) / jnp.maximum(ones * c_str, ones)
        loss = (s_hi + borrow) / kf
        obuf[...] = loss
        pltpu.sync_copy(obuf, loss_hbm)


_pass1 = functools.partial(
    pl.kernel,
    out_type=[jax.ShapeDtypeStruct((NW * NB,), jnp.float32),
              jax.ShapeDtypeStruct((NW * NB,), jnp.float32)],
    mesh=_mesh,
    compiler_params=_cparams,
    scratch_types=[pltpu.VMEM((CH,), jnp.float32),
                   pltpu.VMEM((CH,), jnp.float32),
                   pltpu.VMEM((NB * L,), jnp.float32),
                   pltpu.VMEM((NB * L,), jnp.float32),
                   pltpu.VMEM((NB,), jnp.float32),
                   pltpu.VMEM((NB,), jnp.float32)],
)(_pass1_body)

_pass2 = functools.partial(
    pl.kernel,
    out_type=[jax.ShapeDtypeStruct((NW * NB,), jnp.float32),
              jax.ShapeDtypeStruct((NW * NB,), jnp.float32)],
    mesh=_mesh,
    compiler_params=_cparams,
    scratch_types=[pltpu.VMEM((CH,), jnp.float32),
                   pltpu.VMEM((CH,), jnp.float32),
                   pltpu.VMEM((NB * L,), jnp.float32),
                   pltpu.VMEM((NB * L,), jnp.float32),
                   pltpu.VMEM((NB,), jnp.float32),
                   pltpu.VMEM((NB,), jnp.float32)],
)(_pass2_body)

_pass3 = functools.partial(
    pl.kernel,
    out_type=jax.ShapeDtypeStruct((L,), jnp.float32),
    mesh=_mesh,
    compiler_params=_cparams,
    scratch_types=[pltpu.VMEM((CH,), jnp.float32),
                   pltpu.VMEM((NB,), jnp.float32),
                   pltpu.VMEM((NB,), jnp.float32),
                   pltpu.VMEM((L,), jnp.float32)],
)(_pass3_body)


def kernel(task_score_head, task_score_labels, task_agn_idx):
    del task_agn_idx
    x = task_score_head.reshape(N)
    y = task_score_labels.reshape(N)
    h1c, h1s = _pass1(x, y)
    h2c, h2s = _pass2(x, y, h1c)
    loss_vec = _pass3(h1c, h1s, h2c, h2s)
    return loss_vec[0]


# speedup vs baseline: 5.9882x; 2.2129x over previous
import functools

import jax
import jax.numpy as jnp
from jax import lax
from jax.experimental import pallas as pl
from jax.experimental.pallas import tpu as pltpu
from jax.experimental.pallas import tpu_sc as plsc

N = 1048576
K = 4096
NC = 2
NS = 16
NW = NC * NS
L = 16
M = N // NW
CH = 8192
NCH = M // CH
NB = 2048
NBV = NB // L
UNROLL = 8

_mesh = plsc.VectorSubcoreMesh(core_axis_name="c", subcore_axis_name="s")
_cparams = pltpu.CompilerParams(needs_layout_passes=False)


def _bce16(x, y):
    e = jnp.exp(-jnp.abs(x))
    z = e / (e + 2.0)
    p = z * z
    l1p = 2.0 * z * (1.0 + p * (0.33333333 + p * (0.2 + p * 0.14285714)))
    return jnp.maximum(x, 0.0) - x * y + l1p


def _keybins(bce):
    key = plsc.bitcast(bce, jnp.int32)
    sh20 = jnp.full((L,), 20, jnp.int32)
    sh9 = jnp.full((L,), 9, jnp.int32)
    m11 = jnp.full((L,), 0x7FF, jnp.int32)
    b1 = lax.shift_right_logical(key, sh20)
    b2 = jnp.bitwise_and(lax.shift_right_logical(key, sh9), m11)
    return b1, b2


def _zero_hists(hc, hs):
    zero16 = jnp.zeros((L,), jnp.float32)

    def z(i, _):
        hc[pl.ds(i * L, L)] = zero16
        hs[pl.ds(i * L, L)] = zero16
        return 0

    lax.fori_loop(0, NB * L // L, z, 0, unroll=8)


def _merge_lanes(hc, hs, mc, ms):
    def merge(j, _):
        accc = jnp.zeros((L,), jnp.float32)
        accs = jnp.zeros((L,), jnp.float32)
        for lane in range(L):
            accc = accc + hc[pl.ds(lane * NB + j * L, L)]
            accs = accs + hs[pl.ds(lane * NB + j * L, L)]
        mc[pl.ds(j * L, L)] = accc
        ms[pl.ds(j * L, L)] = accs
        return 0

    lax.fori_loop(0, NBV, merge, 0)


def _core_merge_and_store(hc, hs, mc, ms, shc, shs, out_c, out_s, cid, sid):
    pltpu.sync_copy(mc, shc.at[pl.ds(sid * NB, NB)])
    pltpu.sync_copy(ms, shs.at[pl.ds(sid * NB, NB)])
    plsc.subcore_barrier()

    @pl.when(sid == 0)
    def _():
        pltpu.sync_copy(shc, hc)
        pltpu.sync_copy(shs, hs)
        _merge_lanes(hc, hs, mc, ms)
        pltpu.sync_copy(mc, out_c.at[pl.ds(cid * NB, NB)])
        pltpu.sync_copy(ms, out_s.at[pl.ds(cid * NB, NB)])


def _find_bin(mc, threshold):
    def body(jj, carry):
        cnt_acc, sum_carry = carry
        j = NBV - 1 - jj
        v = mc[pl.ds(j * L, L)]
        sfx = lax.rev(jnp.cumsum(lax.rev(v, (0,))), (0,)) + sum_carry
        ge = sfx >= threshold
        cnt_acc = cnt_acc + plsc.all_reduce_population_count(ge)
        return cnt_acc, sum_carry + jnp.sum(v)

    cnt, _ = lax.fori_loop(
        0, NBV, body, (jnp.zeros((L,), jnp.int32), jnp.float32(0.0))
    )
    return cnt - 1


def _merge_two_rows(src_hbm, buf, dst):
    pltpu.sync_copy(src_hbm, buf)

    def acc(j, _):
        dst[pl.ds(j * L, L)] = buf[pl.ds(j * L, L)] + buf[pl.ds(NB + j * L, L)]
        return 0

    lax.fori_loop(0, NBV, acc, 0, unroll=4)


def _pass1_body(x_hbm, y_hbm, bce_hbm, h1c_hbm, h1s_hbm,
                xbuf, ybuf, bbuf, hc, hs, mc, ms, shc, shs):
    cid = lax.axis_index("c")
    sid = lax.axis_index("s")
    wid = sid * NC + cid
    base = wid * M
    lane = lax.iota(jnp.int32, L)
    ones = jnp.ones((L,), jnp.float32)
    _zero_hists(hc, hs)
    for ch in range(NCH):
        pltpu.sync_copy(x_hbm.at[pl.ds(base + ch * CH, CH)], xbuf)
        pltpu.sync_copy(y_hbm.at[pl.ds(base + ch * CH, CH)], ybuf)

        def body(i, _):
            x = xbuf[pl.ds(i * L, L)]
            y = ybuf[pl.ds(i * L, L)]
            bce = _bce16(x, y)
            bbuf[pl.ds(i * L, L)] = bce
            b1, _b2 = _keybins(bce)
            idx = b1 + lane * NB
            plsc.addupdate_scatter(hc, [idx], ones)
            plsc.addupdate_scatter(hs, [idx], bce)
            return 0

        lax.fori_loop(0, CH // L, body, 0, unroll=UNROLL)
        pltpu.sync_copy(bbuf, bce_hbm.at[pl.ds(base + ch * CH, CH)])
    _merge_lanes(hc, hs, mc, ms)
    _core_merge_and_store(hc, hs, mc, ms, shc, shs, h1c_hbm, h1s_hbm, cid, sid)


def _pass2_body(bce_hbm, h1c_hbm, h2c_hbm, h2s_hbm,
                bbuf, tbuf, hc, hs, mc, ms, shc, shs):
    cid = lax.axis_index("c")
    sid = lax.axis_index("s")
    wid = sid * NC + cid
    base = wid * M
    lane = lax.iota(jnp.int32, L)
    ones = jnp.ones((L,), jnp.float32)
    _merge_two_rows(h1c_hbm, tbuf, mc)
    b1_splat = _find_bin(mc, jnp.float32(float(K)))
    _zero_hists(hc, hs)
    for ch in range(NCH):
        pltpu.sync_copy(bce_hbm.at[pl.ds(base + ch * CH, CH)], bbuf)

        def body(i, _):
            bce = bbuf[pl.ds(i * L, L)]
            b1, b2 = _keybins(bce)
            mask = b1 == b1_splat
            idx = b2 + lane * NB
            plsc.addupdate_scatter(hc, [idx], ones, mask=mask)
            plsc.addupdate_scatter(hs, [idx], bce, mask=mask)
            return 0

        lax.fori_loop(0, CH // L, body, 0, unroll=UNROLL)
    _merge_lanes(hc, hs, mc, ms)
    _core_merge_and_store(hc, hs, mc, ms, shc, shs, h2c_hbm, h2s_hbm, cid, sid)


def _masked_sums(mc, ms, bin_splat):
    lane = lax.iota(jnp.int32, L)

    def body(j, carry):
        cgt, sgt, ceq, seq = carry
        ids = lane + j * L
        vc = mc[pl.ds(j * L, L)]
        vs = ms[pl.ds(j * L, L)]
        gt = ids > bin_splat
        eq = ids == bin_splat
        zero = jnp.zeros((L,), jnp.float32)
        return (cgt + jnp.where(gt, vc, zero), sgt + jnp.where(gt, vs, zero),
                ceq + jnp.where(eq, vc, zero), seq + jnp.where(eq, vs, zero))

    z = jnp.zeros((L,), jnp.float32)
    cgt, sgt, ceq, seq = lax.fori_loop(0, NBV, body, (z, z, z, z))
    return jnp.sum(cgt), jnp.sum(sgt), jnp.sum(ceq), jnp.sum(seq)


def _pass3_body(h1c_hbm, h1s_hbm, h2c_hbm, h2s_hbm, loss_hbm,
                tbuf, mc, ms, obuf):
    cid = lax.axis_index("c")
    sid = lax.axis_index("s")

    @pl.when((sid == 0) & (cid == 0))
    def _():
        _merge_two_rows(h1c_hbm, tbuf, mc)
        _merge_two_rows(h1s_hbm, tbuf, ms)
        b1_splat = _find_bin(mc, jnp.float32(float(K)))
        c_ab, s_ab, _c1, _s1 = _masked_sums(mc, ms, b1_splat)
        _merge_two_rows(h2c_hbm, tbuf, mc)
        _merge_two_rows(h2s_hbm, tbuf, ms)
        t2 = jnp.float32(float(K)) - c_ab
        b2_splat = _find_bin(mc, t2)
        c_hi2, s_hi2, c_str, s_str = _masked_sums(mc, ms, b2_splat)
        ones = jnp.ones((L,), jnp.float32)
        kf = jnp.full((L,), float(K), jnp.float32)
        c_hi = ones * c_ab + ones * c_hi2
        s_hi = ones * s_ab + ones * s_hi2
        borrow = (kf - c_hi) * (ones * s_str) / jnp.maximum(ones * c_str, ones)
        loss = (s_hi + borrow) / kf
        obuf[...] = loss
        pltpu.sync_copy(obuf, loss_hbm)


_pass1 = functools.partial(
    pl.kernel,
    out_type=[jax.ShapeDtypeStruct((N,), jnp.float32),
              jax.ShapeDtypeStruct((NC * NB,), jnp.float32),
              jax.ShapeDtypeStruct((NC * NB,), jnp.float32)],
    mesh=_mesh,
    compiler_params=_cparams,
    scratch_types=[pltpu.VMEM((CH,), jnp.float32),
                   pltpu.VMEM((CH,), jnp.float32),
                   pltpu.VMEM((CH,), jnp.float32),
                   pltpu.VMEM((NB * L,), jnp.float32),
                   pltpu.VMEM((NB * L,), jnp.float32),
                   pltpu.VMEM((NB,), jnp.float32),
                   pltpu.VMEM((NB,), jnp.float32),
                   pltpu.VMEM_SHARED((NS * NB,), jnp.float32),
                   pltpu.VMEM_SHARED((NS * NB,), jnp.float32)],
)(_pass1_body)

_pass2 = functools.partial(
    pl.kernel,
    out_type=[jax.ShapeDtypeStruct((NC * NB,), jnp.float32),
              jax.ShapeDtypeStruct((NC * NB,), jnp.float32)],
    mesh=_mesh,
    compiler_params=_cparams,
    scratch_types=[pltpu.VMEM((CH,), jnp.float32),
                   pltpu.VMEM((NC * NB,), jnp.float32),
                   pltpu.VMEM((NB * L,), jnp.float32),
                   pltpu.VMEM((NB * L,), jnp.float32),
                   pltpu.VMEM((NB,), jnp.float32),
                   pltpu.VMEM((NB,), jnp.float32),
                   pltpu.VMEM_SHARED((NS * NB,), jnp.float32),
                   pltpu.VMEM_SHARED((NS * NB,), jnp.float32)],
)(_pass2_body)

_pass3 = functools.partial(
    pl.kernel,
    out_type=jax.ShapeDtypeStruct((L,), jnp.float32),
    mesh=_mesh,
    compiler_params=_cparams,
    scratch_types=[pltpu.VMEM((NC * NB,), jnp.float32),
                   pltpu.VMEM((NB,), jnp.float32),
                   pltpu.VMEM((NB,), jnp.float32),
                   pltpu.VMEM((L,), jnp.float32)],
)(_pass3_body)


def kernel(task_score_head, task_score_labels, task_agn_idx):
    del task_agn_idx
    x = task_score_head.reshape(N)
    y = task_score_labels.reshape(N)
    bce, h1c, h1s = _pass1(x, y)
    h2c, h2s = _pass2(bce, h1c)
    loss_vec = _pass3(h1c, h1s, h2c, h2s)
    return loss_vec[0]


# speedup vs baseline: 12.1050x; 2.0215x over previous
import functools

import jax
import jax.numpy as jnp
from jax import lax
from jax.experimental import pallas as pl
from jax.experimental.pallas import tpu as pltpu
from jax.experimental.pallas import tpu_sc as plsc

N = 1048576
K = 4096
NC = 2
NS = 16
NW = NC * NS
L = 16
M = N // NW
CH = 8192
NCH = M // CH
NB = 2048
NBV = NB // L
U = 8

_mesh = plsc.VectorSubcoreMesh(core_axis_name="c", subcore_axis_name="s")
_cparams = pltpu.CompilerParams(needs_layout_passes=False)


def _bce16(x, y):
    e = jnp.exp(-jnp.abs(x))
    z = e / (e + 2.0)
    p = z * z
    l1p = 2.0 * z * (1.0 + p * (0.33333333 + p * (0.2 + p * 0.14285714)))
    return jnp.maximum(x, 0.0) - x * y + l1p


def _keybins(bce):
    key = plsc.bitcast(bce, jnp.int32)
    sh20 = jnp.full((L,), 20, jnp.int32)
    sh9 = jnp.full((L,), 9, jnp.int32)
    m11 = jnp.full((L,), 0x7FF, jnp.int32)
    b1 = lax.shift_right_logical(key, sh20)
    b2 = jnp.bitwise_and(lax.shift_right_logical(key, sh9), m11)
    return b1, b2


def _zero_hists(hc, hs):
    zero16 = jnp.zeros((L,), jnp.float32)

    def z(i, _):
        hc[pl.ds(i * L, L)] = zero16
        hs[pl.ds(i * L, L)] = zero16
        return 0

    lax.fori_loop(0, NB * L // L, z, 0, unroll=8)


def _merge_lanes(hc, hs, mc, ms):
    def merge(j, _):
        accc = jnp.zeros((L,), jnp.float32)
        accs = jnp.zeros((L,), jnp.float32)
        for lane in range(L):
            accc = accc + hc[pl.ds(lane * NB + j * L, L)]
            accs = accs + hs[pl.ds(lane * NB + j * L, L)]
        mc[pl.ds(j * L, L)] = accc
        ms[pl.ds(j * L, L)] = accs
        return 0

    lax.fori_loop(0, NBV, merge, 0)


def _core_merge_and_store(hc, hs, mc, ms, shc, shs, out_c, out_s, cid, sid):
    pltpu.sync_copy(mc, shc.at[pl.ds(sid * NB, NB)])
    pltpu.sync_copy(ms, shs.at[pl.ds(sid * NB, NB)])
    plsc.subcore_barrier()

    @pl.when(sid == 0)
    def _():
        pltpu.sync_copy(shc, hc)
        pltpu.sync_copy(shs, hs)
        _merge_lanes(hc, hs, mc, ms)
        pltpu.sync_copy(mc, out_c.at[pl.ds(cid * NB, NB)])
        pltpu.sync_copy(ms, out_s.at[pl.ds(cid * NB, NB)])


def _find_bin(mc, threshold):
    def body(jj, carry):
        cnt_acc, sum_carry = carry
        j = NBV - 1 - jj
        v = mc[pl.ds(j * L, L)]
        sfx = lax.rev(jnp.cumsum(lax.rev(v, (0,))), (0,)) + sum_carry
        ge = sfx >= threshold
        cnt_acc = cnt_acc + plsc.all_reduce_population_count(ge)
        return cnt_acc, sum_carry + jnp.sum(v)

    cnt, _ = lax.fori_loop(
        0, NBV, body, (jnp.zeros((L,), jnp.int32), jnp.float32(0.0))
    )
    return cnt - 1


def _merge_two_rows(src_hbm, buf, dst):
    pltpu.sync_copy(src_hbm, buf)

    def acc(j, _):
        dst[pl.ds(j * L, L)] = buf[pl.ds(j * L, L)] + buf[pl.ds(NB + j * L, L)]
        return 0

    lax.fori_loop(0, NBV, acc, 0, unroll=4)


def _pass1_body(x_hbm, y_hbm, bce_hbm, h1c_hbm, h1s_hbm,
                xbuf0, ybuf0, bbuf0, xbuf1, ybuf1, bbuf1,
                hc, hs, mc, ms, shc, shs, semi0, semi1, semo0, semo1):
    cid = lax.axis_index("c")
    sid = lax.axis_index("s")
    wid = sid * NC + cid
    base = wid * M
    lane = lax.iota(jnp.int32, L)
    ones = jnp.ones((L,), jnp.float32)
    bufs = [(xbuf0, ybuf0, bbuf0, semi0, semo0),
            (xbuf1, ybuf1, bbuf1, semi1, semo1)]
    descs_in = [None, None]
    descs_out = [None, None]

    def start_in(ch):
        p = ch & 1
        xb, yb, _bb, semi, _semo = bufs[p]
        dx = pltpu.async_copy(x_hbm.at[pl.ds(base + ch * CH, CH)], xb, semi)
        dy = pltpu.async_copy(y_hbm.at[pl.ds(base + ch * CH, CH)], yb, semi)
        descs_in[p] = (dx, dy)

    start_in(0)
    _zero_hists(hc, hs)
    for ch in range(NCH):
        p = ch & 1
        xb, yb, bb, _semi, semo = bufs[p]
        dx, dy = descs_in[p]
        dx.wait()
        dy.wait()
        if ch + 1 < NCH:
            start_in(ch + 1)
        if descs_out[p] is not None:
            descs_out[p].wait()

        def body(i, _):
            off = i * (U * L)
            xs = [xb[pl.ds(off + u * L, L)] for u in range(U)]
            ys = [yb[pl.ds(off + u * L, L)] for u in range(U)]
            es = [jnp.exp(-jnp.abs(x)) for x in xs]
            zs = [e / (e + 2.0) for e in es]
            ps = [z * z for z in zs]
            l1 = [2.0 * z * (1.0 + p2 * (0.33333333 + p2 * (0.2 + p2 * 0.14285714)))
                  for z, p2 in zip(zs, ps)]
            bces = [jnp.maximum(x, 0.0) - x * y + l
                    for x, y, l in zip(xs, ys, l1)]
            for u in range(U):
                bb[pl.ds(off + u * L, L)] = bces[u]
            for u in range(U):
                b1, _b2 = _keybins(bces[u])
                idx = b1 + lane * NB
                plsc.addupdate_scatter(hc, [idx], ones)
                plsc.addupdate_scatter(hs, [idx], bces[u])
            return 0

        lax.fori_loop(0, CH // (U * L), body, 0)
        descs_out[p] = pltpu.async_copy(
            bb, bce_hbm.at[pl.ds(base + ch * CH, CH)], semo)
    for p in range(2):
        if descs_out[p] is not None:
            descs_out[p].wait()
    _merge_lanes(hc, hs, mc, ms)
    _core_merge_and_store(hc, hs, mc, ms, shc, shs, h1c_hbm, h1s_hbm, cid, sid)


def _pass2_body(bce_hbm, h1c_hbm, h2c_hbm, h2s_hbm,
                bbuf0, bbuf1, tbuf, hc, hs, mc, ms, shc, shs, semi0, semi1):
    cid = lax.axis_index("c")
    sid = lax.axis_index("s")
    wid = sid * NC + cid
    base = wid * M
    lane = lax.iota(jnp.int32, L)
    ones = jnp.ones((L,), jnp.float32)
    bufs = [(bbuf0, semi0), (bbuf1, semi1)]
    descs_in = [None, None]

    def start_in(ch):
        p = ch & 1
        bb, semi = bufs[p]
        descs_in[p] = pltpu.async_copy(
            bce_hbm.at[pl.ds(base + ch * CH, CH)], bb, semi)

    start_in(0)
    _merge_two_rows(h1c_hbm, tbuf, mc)
    b1_splat = _find_bin(mc, jnp.float32(float(K)))
    _zero_hists(hc, hs)
    for ch in range(NCH):
        p = ch & 1
        bb, _semi = bufs[p]
        descs_in[p].wait()
        if ch + 1 < NCH:
            start_in(ch + 1)

        def body(i, _):
            off = i * (U * L)
            bces = [bb[pl.ds(off + u * L, L)] for u in range(U)]
            for u in range(U):
                b1, b2 = _keybins(bces[u])
                mask = b1 == b1_splat
                idx = b2 + lane * NB
                plsc.addupdate_scatter(hc, [idx], ones, mask=mask)
                plsc.addupdate_scatter(hs, [idx], bces[u], mask=mask)
            return 0

        lax.fori_loop(0, CH // (U * L), body, 0)
    _merge_lanes(hc, hs, mc, ms)
    _core_merge_and_store(hc, hs, mc, ms, shc, shs, h2c_hbm, h2s_hbm, cid, sid)


def _masked_sums(mc, ms, bin_splat):
    lane = lax.iota(jnp.int32, L)

    def body(j, carry):
        cgt, sgt, ceq, seq = carry
        ids = lane + j * L
        vc = mc[pl.ds(j * L, L)]
        vs = ms[pl.ds(j * L, L)]
        gt = ids > bin_splat
        eq = ids == bin_splat
        zero = jnp.zeros((L,), jnp.float32)
        return (cgt + jnp.where(gt, vc, zero), sgt + jnp.where(gt, vs, zero),
                ceq + jnp.where(eq, vc, zero), seq + jnp.where(eq, vs, zero))

    z = jnp.zeros((L,), jnp.float32)
    cgt, sgt, ceq, seq = lax.fori_loop(0, NBV, body, (z, z, z, z))
    return jnp.sum(cgt), jnp.sum(sgt), jnp.sum(ceq), jnp.sum(seq)


def _pass3_body(h1c_hbm, h1s_hbm, h2c_hbm, h2s_hbm, loss_hbm,
                tbuf, mc, ms, obuf):
    cid = lax.axis_index("c")
    sid = lax.axis_index("s")

    @pl.when((sid == 0) & (cid == 0))
    def _():
        _merge_two_rows(h1c_hbm, tbuf, mc)
        _merge_two_rows(h1s_hbm, tbuf, ms)
        b1_splat = _find_bin(mc, jnp.float32(float(K)))
        c_ab, s_ab, _c1, _s1 = _masked_sums(mc, ms, b1_splat)
        _merge_two_rows(h2c_hbm, tbuf, mc)
        _merge_two_rows(h2s_hbm, tbuf, ms)
        t2 = jnp.float32(float(K)) - c_ab
        b2_splat = _find_bin(mc, t2)
        c_hi2, s_hi2, c_str, s_str = _masked_sums(mc, ms, b2_splat)
        ones = jnp.ones((L,), jnp.float32)
        kf = jnp.full((L,), float(K), jnp.float32)
        c_hi = ones * c_ab + ones * c_hi2
        s_hi = ones * s_ab + ones * s_hi2
        borrow = (kf - c_hi) * (ones * s_str) / jnp.maximum(ones * c_str, ones)
        loss = (s_hi + borrow) / kf
        obuf[...] = loss
        pltpu.sync_copy(obuf, loss_hbm)


_pass1 = functools.partial(
    pl.kernel,
    out_type=[jax.ShapeDtypeStruct((N,), jnp.float32),
              jax.ShapeDtypeStruct((NC * NB,), jnp.float32),
              jax.ShapeDtypeStruct((NC * NB,), jnp.float32)],
    mesh=_mesh,
    compiler_params=_cparams,
    scratch_types=[pltpu.VMEM((CH,), jnp.float32),
                   pltpu.VMEM((CH,), jnp.float32),
                   pltpu.VMEM((CH,), jnp.float32),
                   pltpu.VMEM((CH,), jnp.float32),
                   pltpu.VMEM((CH,), jnp.float32),
                   pltpu.VMEM((CH,), jnp.float32),
                   pltpu.VMEM((NB * L,), jnp.float32),
                   pltpu.VMEM((NB * L,), jnp.float32),
                   pltpu.VMEM((NB,), jnp.float32),
                   pltpu.VMEM((NB,), jnp.float32),
                   pltpu.VMEM_SHARED((NS * NB,), jnp.float32),
                   pltpu.VMEM_SHARED((NS * NB,), jnp.float32),
                   pltpu.SemaphoreType.DMA,
                   pltpu.SemaphoreType.DMA,
                   pltpu.SemaphoreType.DMA,
                   pltpu.SemaphoreType.DMA],
)(_pass1_body)

_pass2 = functools.partial(
    pl.kernel,
    out_type=[jax.ShapeDtypeStruct((NC * NB,), jnp.float32),
              jax.ShapeDtypeStruct((NC * NB,), jnp.float32)],
    mesh=_mesh,
    compiler_params=_cparams,
    scratch_types=[pltpu.VMEM((CH,), jnp.float32),
                   pltpu.VMEM((CH,), jnp.float32),
                   pltpu.VMEM((NC * NB,), jnp.float32),
                   pltpu.VMEM((NB * L,), jnp.float32),
                   pltpu.VMEM((NB * L,), jnp.float32),
                   pltpu.VMEM((NB,), jnp.float32),
                   pltpu.VMEM((NB,), jnp.float32),
                   pltpu.VMEM_SHARED((NS * NB,), jnp.float32),
                   pltpu.VMEM_SHARED((NS * NB,), jnp.float32),
                   pltpu.SemaphoreType.DMA,
                   pltpu.SemaphoreType.DMA],
)(_pass2_body)

_pass3 = functools.partial(
    pl.kernel,
    out_type=jax.ShapeDtypeStruct((L,), jnp.float32),
    mesh=_mesh,
    compiler_params=_cparams,
    scratch_types=[pltpu.VMEM((NC * NB,), jnp.float32),
                   pltpu.VMEM((NB,), jnp.float32),
                   pltpu.VMEM((NB,), jnp.float32),
                   pltpu.VMEM((L,), jnp.float32)],
)(_pass3_body)


def kernel(task_score_head, task_score_labels, task_agn_idx):
    del task_agn_idx
    x = task_score_head.reshape(N)
    y = task_score_labels.reshape(N)
    bce, h1c, h1s = _pass1(x, y)
    h2c, h2s = _pass2(bce, h1c)
    loss_vec = _pass3(h1c, h1s, h2c, h2s)
    return loss_vec[0]
